# baseline (speedup 1.0000x reference)
import dataclasses

import jax
import jax.numpy as jnp
from jax import lax
from jax.experimental import pallas as pl
from jax.experimental.pallas import tpu as pltpu
from jax.experimental.pallas import tpu_sc as plsc

SEQ = 1000000
NM = 6
NMATCH = 50000
PLEN = 31
HALF = 15

PAD = 1048576
NC, NS = 2, 16
NW = NC * NS
SEG = PAD // NW
CHUNKI = 3968
NFULL = 12
TAIL = NMATCH - NFULL * CHUNKI


@jax.jit
def _sc_scatter(pos, probs):
    mesh = plsc.VectorSubcoreMesh(
        core_axis_name="c", subcore_axis_name="s",
        num_cores=NC, num_subcores=NS)

    @pl.kernel(
        out_type=[jax.ShapeDtypeStruct((1024, 1024), jnp.float32),
                  jax.ShapeDtypeStruct((NW, 16), jnp.float32)],
        mesh=mesh,
        scratch_types=[
            pltpu.VMEM((SEG,), jnp.float32),
            pltpu.VMEM((2, CHUNKI), jnp.int32),
            pltpu.VMEM((TAIL,), jnp.int32),
            pltpu.VMEM((NM, 16), jnp.float32),
            pltpu.VMEM((1, 16), jnp.float32),
            pltpu.SemaphoreType.DMA,
            pltpu.SemaphoreType.DMA,
            pltpu.SemaphoreType.DMA,
        ],
        compiler_params=dataclasses.replace(
            pltpu.CompilerParams(), needs_layout_passes=False),
    )
    def body(pos_hbm, probs_hbm, out_hbm, psum_hbm, seg, ibuf, tbuf, pbuf,
             abuf, sem0, sem1, sem2):
        c = lax.axis_index("c")
        s = lax.axis_index("s")
        wid = c * NS + s
        base = wid * SEG
        lo = jnp.zeros((16,), jnp.int32) + base
        seg_u = jnp.full((16,), SEG, jnp.uint32)
        zero16 = jnp.zeros((16,), jnp.float32)

        pltpu.async_copy(probs_hbm, pbuf, sem1)

        @plsc.parallel_loop(0, SEG, step=128, unroll=2)
        def _(j):
            for t in range(8):
                seg[pl.ds(j + t * 16, 16)] = zero16

        pltpu.make_async_copy(probs_hbm, pbuf, sem1).wait()

        @pl.loop(0, NM)
        def _(i):
            pv = pbuf[i, :]

            def chunk_ref(k, i=i):
                return pos_hbm.at[i, pl.ds(k * CHUNKI, CHUNKI)]

            def scat(slot, j, off, pv=pv):
                lv = ibuf[slot, pl.ds(j + off, 16)] - lo
                keep = plsc.bitcast(lv, jnp.uint32) < seg_u
                plsc.store_scatter(seg, [lv], pv, mask=keep)

            def process(slot, pv=pv):
                @plsc.parallel_loop(0, CHUNKI, step=128, unroll=2)
                def _(j):
                    for t in range(8):
                        scat(slot, j, t * 16)

            def scat_t(j, off, pv=pv):
                lv = tbuf[pl.ds(j + off, 16)] - lo
                keep = plsc.bitcast(lv, jnp.uint32) < seg_u
                plsc.store_scatter(seg, [lv], pv, mask=keep)

            tail_ref = pos_hbm.at[i, pl.ds(NFULL * CHUNKI, TAIL)]

            pltpu.async_copy(chunk_ref(0), ibuf.at[0], sem0)
            pltpu.async_copy(tail_ref, tbuf, sem2)

            @pl.loop(0, NFULL, step=2)
            def _(k):
                pltpu.async_copy(chunk_ref(k + 1), ibuf.at[1], sem1)
                pltpu.make_async_copy(chunk_ref(k), ibuf.at[0], sem0).wait()
                process(0)

                @pl.when(k + 2 < NFULL)
                def _():
                    pltpu.async_copy(chunk_ref(k + 2), ibuf.at[0], sem0)

                pltpu.make_async_copy(chunk_ref(k + 1), ibuf.at[1],
                                      sem1).wait()
                process(1)

            pltpu.make_async_copy(tail_ref, tbuf, sem2).wait()

            @plsc.parallel_loop(0, TAIL - TAIL % 128, step=128, unroll=2)
            def _(j):
                for t in range(8):
                    scat_t(j, t * 16)

            for t in range(TAIL % 128 // 16):
                scat_t(TAIL - TAIL % 128, t * 16)

        @pl.loop(0, SEG // 1024)
        def _(r):
            pltpu.async_copy(seg.at[pl.ds(r * 1024, 1024)],
                             out_hbm.at[wid * (SEG // 1024) + r], sem0)

        def sum_step(j, accs):
            a0, a1 = accs
            a0 = a0 + seg[pl.ds(j, 16)] + seg[pl.ds(j + 32, 16)]
            a1 = a1 + seg[pl.ds(j + 16, 16)] + seg[pl.ds(j + 48, 16)]
            return a0, a1

        a0, a1 = lax.fori_loop(
            0, SEG // 64, lambda t, accs: sum_step(t * 64, accs),
            (zero16, zero16))
        abuf[0, :] = a0 + a1
        pltpu.sync_copy(abuf.at[0], psum_hbm.at[wid])

        @pl.loop(0, SEG // 1024)
        def _(r):
            pltpu.make_async_copy(seg.at[pl.ds(r * 1024, 1024)],
                                  out_hbm.at[wid * (SEG // 1024) + r],
                                  sem0).wait()

    return body(pos, probs)


NB = 8
BR = 1024 // NB


def _dense_body(cur_ref, prev_ref, next_ref, head_ref, tail_ref, psum_ref,
                w_ref, prof_ref, rp_ref, sp_ref, out1_ref, out2_ref):
    i = pl.program_id(0)
    cur = cur_ref[...]

    row_above = prev_ref[7:8, :]
    row_above = jnp.where(i == 0, jnp.float32(0.0), row_above)
    row_below = next_ref[0:1, :]
    up = jnp.concatenate([row_above, cur[:-1, :]], axis=0)
    down = jnp.concatenate([cur[1:, :], row_below], axis=0)
    ext = jnp.concatenate(
        [up[:, 1024 - HALF:], cur, down[:, :HALF],
         jnp.zeros((BR, 1152 - 1024 - 2 * HALF), jnp.float32)],
        axis=1).astype(jnp.bfloat16)

    psum = jnp.float32(0.0)
    for k in range(PLEN):
        psum = psum + prof_ref[k]

    w = w_ref[...]
    lp_tiles = []
    for t in range(1024 // 128):
        win = lax.slice(ext, (0, t * 128), (BR, t * 128 + 256))
        lp_tiles.append(jnp.dot(win, w, preferred_element_type=jnp.float32))
    lp = jnp.concatenate(lp_tiles, axis=1)

    s_val = jnp.sum(psum_ref[...])
    corr_term = jnp.float32(0.0)
    for k in range(HALF):
        suf = jnp.sum(lax.slice(tail_ref[...], (0, 576 - (HALF - k)), (1, 576)))
        corr_term = corr_term + prof_ref[k] * suf
    for k in range(HALF + 1, PLEN):
        pre = jnp.sum(lax.slice(head_ref[...], (0, 0), (1, k - HALF)))
        corr_term = corr_term + prof_ref[k] * pre
    c_val = psum * s_val - corr_term

    rp = rp_ref[0]
    sp = sp_ref[0]
    inv_norm = jnp.where(c_val > 0, 1.0 / c_val, 1.0 / s_val)
    a = (rp + (1.0 - rp) * sp) / s_val
    b = (1.0 - rp) * (1.0 - sp) * inv_norm

    out1_ref[...] = a * cur + b * lp
    out2_ref[...] = jnp.where(cur > 0, rp, jnp.float32(0.0))


@jax.jit
def _tc_dense(u2d, psum, profile, rp, sp):
    r = jnp.arange(256, dtype=jnp.int32)[:, None]
    c = jnp.arange(128, dtype=jnp.int32)[None, :]
    kd = r - c
    wmat = jnp.zeros((256, 128), jnp.float32)
    for k in range(PLEN):
        wmat = wmat + jnp.where(kd == k, profile[k], jnp.float32(0.0))
    wmat = wmat.astype(jnp.bfloat16)

    out1, out2 = pl.pallas_call(
        _dense_body,
        grid=(NB,),
        in_specs=[
            pl.BlockSpec((BR, 1024), lambda i: (i, 0)),
            pl.BlockSpec((8, 1024),
                         lambda i: (jnp.maximum(i * (BR // 8) - 1, 0), 0)),
            pl.BlockSpec((8, 1024),
                         lambda i: (jnp.minimum((i + 1) * (BR // 8), 127), 0)),
            pl.BlockSpec((8, 1024), lambda i: (0, 0)),
            pl.BlockSpec((8, 1024), lambda i: (122, 0)),
            pl.BlockSpec((NW, 16), lambda i: (0, 0)),
            pl.BlockSpec((256, 128), lambda i: (0, 0)),
            pl.BlockSpec(memory_space=pltpu.SMEM),
            pl.BlockSpec(memory_space=pltpu.SMEM),
            pl.BlockSpec(memory_space=pltpu.SMEM),
        ],
        out_specs=[
            pl.BlockSpec((BR, 1024), lambda i: (i, 0)),
            pl.BlockSpec((BR, 1024), lambda i: (i, 0)),
        ],
        out_shape=[
            jax.ShapeDtypeStruct((1024, 1024), jnp.float32),
            jax.ShapeDtypeStruct((1024, 1024), jnp.float32),
        ],
    )(u2d, u2d, u2d, u2d, u2d, psum, wmat, profile, rp, sp)
    return out1, out2


def kernel(positions, motifs_prob, profile, replication_prob, short_patch_ber_prob):
    probs_b = jnp.broadcast_to(motifs_prob[:, None], (NM, 16))

    u, psum = _sc_scatter(positions, probs_b)
    out1, out2 = _tc_dense(u, psum, profile,
                           replication_prob, short_patch_ber_prob)
    return (out1.reshape(-1)[:SEQ], out2.reshape(-1)[:SEQ])


# speedup vs baseline: 1.0326x; 1.0326x over previous
import dataclasses

import jax
import jax.numpy as jnp
from jax import lax
from jax.experimental import pallas as pl
from jax.experimental.pallas import tpu as pltpu
from jax.experimental.pallas import tpu_sc as plsc

SEQ = 1000000
NM = 6
NMATCH = 50000
PLEN = 31
HALF = 15

PAD = 1048576
NC, NS = 2, 16
NW = NC * NS
SEG = PAD // NW
CHUNKI = 3968
NFULL = 12
TAIL = NMATCH - NFULL * CHUNKI


@jax.jit
def _sc_scatter(pos, probs):
    mesh = plsc.VectorSubcoreMesh(
        core_axis_name="c", subcore_axis_name="s",
        num_cores=NC, num_subcores=NS)

    @pl.kernel(
        out_type=[jax.ShapeDtypeStruct((1024, 1024), jnp.float32),
                  jax.ShapeDtypeStruct((NW, 16), jnp.float32)],
        mesh=mesh,
        scratch_types=[
            pltpu.VMEM((SEG,), jnp.float32),
            pltpu.VMEM((2, CHUNKI), jnp.int32),
            pltpu.VMEM((TAIL,), jnp.int32),
            pltpu.VMEM((NM, 16), jnp.float32),
            pltpu.VMEM((1, 16), jnp.float32),
            pltpu.SemaphoreType.DMA,
            pltpu.SemaphoreType.DMA,
            pltpu.SemaphoreType.DMA,
        ],
        compiler_params=dataclasses.replace(
            pltpu.CompilerParams(), needs_layout_passes=False),
    )
    def body(pos_hbm, probs_hbm, out_hbm, psum_hbm, seg, ibuf, tbuf, pbuf,
             abuf, sem0, sem1, sem2):
        c = lax.axis_index("c")
        s = lax.axis_index("s")
        wid = c * NS + s
        base = wid * SEG
        lo = jnp.zeros((16,), jnp.int32) + base
        seg_u = jnp.full((16,), SEG, jnp.uint32)
        zero16 = jnp.zeros((16,), jnp.float32)

        pltpu.async_copy(probs_hbm, pbuf, sem1)

        @plsc.parallel_loop(0, SEG, step=128, unroll=2)
        def _(j):
            for t in range(8):
                seg[pl.ds(j + t * 16, 16)] = zero16

        pltpu.make_async_copy(probs_hbm, pbuf, sem1).wait()

        @pl.loop(0, NM)
        def _(i):
            pv = pbuf[i, :]

            def chunk_ref(k, i=i):
                return pos_hbm.at[i, pl.ds(k * CHUNKI, CHUNKI)]

            def scat(slot, j, off, pv=pv):
                lv = ibuf[slot, pl.ds(j + off, 16)] - lo
                keep = plsc.bitcast(lv, jnp.uint32) < seg_u
                plsc.store_scatter(seg, [lv], pv, mask=keep)

            def process(slot, pv=pv):
                @plsc.parallel_loop(0, CHUNKI, step=128, unroll=2)
                def _(j):
                    for t in range(8):
                        scat(slot, j, t * 16)

            def scat_t(j, off, pv=pv):
                lv = tbuf[pl.ds(j + off, 16)] - lo
                keep = plsc.bitcast(lv, jnp.uint32) < seg_u
                plsc.store_scatter(seg, [lv], pv, mask=keep)

            tail_ref = pos_hbm.at[i, pl.ds(NFULL * CHUNKI, TAIL)]

            pltpu.async_copy(chunk_ref(0), ibuf.at[0], sem0)
            pltpu.async_copy(tail_ref, tbuf, sem2)

            @pl.loop(0, NFULL, step=2)
            def _(k):
                pltpu.async_copy(chunk_ref(k + 1), ibuf.at[1], sem1)
                pltpu.make_async_copy(chunk_ref(k), ibuf.at[0], sem0).wait()
                process(0)

                @pl.when(k + 2 < NFULL)
                def _():
                    pltpu.async_copy(chunk_ref(k + 2), ibuf.at[0], sem0)

                pltpu.make_async_copy(chunk_ref(k + 1), ibuf.at[1],
                                      sem1).wait()
                process(1)

            pltpu.make_async_copy(tail_ref, tbuf, sem2).wait()

            @plsc.parallel_loop(0, TAIL - TAIL % 128, step=128, unroll=2)
            def _(j):
                for t in range(8):
                    scat_t(j, t * 16)

            for t in range(TAIL % 128 // 16):
                scat_t(TAIL - TAIL % 128, t * 16)

        @pl.loop(0, SEG // 1024)
        def _(r):
            pltpu.async_copy(seg.at[pl.ds(r * 1024, 1024)],
                             out_hbm.at[wid * (SEG // 1024) + r], sem0)

        def sum_step(j, accs):
            a0, a1 = accs
            a0 = a0 + seg[pl.ds(j, 16)] + seg[pl.ds(j + 32, 16)]
            a1 = a1 + seg[pl.ds(j + 16, 16)] + seg[pl.ds(j + 48, 16)]
            return a0, a1

        a0, a1 = lax.fori_loop(
            0, SEG // 64, lambda t, accs: sum_step(t * 64, accs),
            (zero16, zero16))
        abuf[0, :] = a0 + a1
        pltpu.sync_copy(abuf.at[0], psum_hbm.at[wid])

        @pl.loop(0, SEG // 1024)
        def _(r):
            pltpu.make_async_copy(seg.at[pl.ds(r * 1024, 1024)],
                                  out_hbm.at[wid * (SEG // 1024) + r],
                                  sem0).wait()

    return body(pos, probs)


NB = 4
BR = 1024 // NB


def _dense_body(cur_ref, prev_ref, next_ref, head_ref, tail_ref, psum_ref,
                w_ref, prof_ref, rp_ref, sp_ref, out1_ref, out2_ref):
    i = pl.program_id(0)
    cur = cur_ref[...]

    row_above = prev_ref[7:8, :]
    row_above = jnp.where(i == 0, jnp.float32(0.0), row_above)
    row_below = next_ref[0:1, :]
    up = jnp.concatenate([row_above, cur[:-1, :]], axis=0)
    down = jnp.concatenate([cur[1:, :], row_below], axis=0)
    ext = jnp.concatenate(
        [up[:, 1024 - HALF:], cur, down[:, :HALF],
         jnp.zeros((BR, 1152 - 1024 - 2 * HALF), jnp.float32)],
        axis=1).astype(jnp.bfloat16)

    psum = jnp.float32(0.0)
    for k in range(PLEN):
        psum = psum + prof_ref[k]

    w = w_ref[...]
    lp_tiles = []
    for t in range(1024 // 128):
        win = lax.slice(ext, (0, t * 128), (BR, t * 128 + 256))
        lp_tiles.append(jnp.dot(win, w, preferred_element_type=jnp.float32))
    lp = jnp.concatenate(lp_tiles, axis=1)

    s_val = jnp.sum(psum_ref[...])
    corr_term = jnp.float32(0.0)
    for k in range(HALF):
        suf = jnp.sum(lax.slice(tail_ref[...], (0, 576 - (HALF - k)), (1, 576)))
        corr_term = corr_term + prof_ref[k] * suf
    for k in range(HALF + 1, PLEN):
        pre = jnp.sum(lax.slice(head_ref[...], (0, 0), (1, k - HALF)))
        corr_term = corr_term + prof_ref[k] * pre
    c_val = psum * s_val - corr_term

    rp = rp_ref[0]
    sp = sp_ref[0]
    inv_norm = jnp.where(c_val > 0, 1.0 / c_val, 1.0 / s_val)
    a = (rp + (1.0 - rp) * sp) / s_val
    b = (1.0 - rp) * (1.0 - sp) * inv_norm

    out1_ref[...] = a * cur + b * lp
    out2_ref[...] = jnp.where(cur > 0, rp, jnp.float32(0.0))


@jax.jit
def _tc_dense(u2d, psum, profile, rp, sp):
    r = jnp.arange(256, dtype=jnp.int32)[:, None]
    c = jnp.arange(128, dtype=jnp.int32)[None, :]
    kd = r - c
    wmat = jnp.zeros((256, 128), jnp.float32)
    for k in range(PLEN):
        wmat = wmat + jnp.where(kd == k, profile[k], jnp.float32(0.0))
    wmat = wmat.astype(jnp.bfloat16)

    out1, out2 = pl.pallas_call(
        _dense_body,
        grid=(NB,),
        in_specs=[
            pl.BlockSpec((BR, 1024), lambda i: (i, 0)),
            pl.BlockSpec((8, 1024),
                         lambda i: (jnp.maximum(i * (BR // 8) - 1, 0), 0)),
            pl.BlockSpec((8, 1024),
                         lambda i: (jnp.minimum((i + 1) * (BR // 8), 127), 0)),
            pl.BlockSpec((8, 1024), lambda i: (0, 0)),
            pl.BlockSpec((8, 1024), lambda i: (122, 0)),
            pl.BlockSpec((NW, 16), lambda i: (0, 0)),
            pl.BlockSpec((256, 128), lambda i: (0, 0)),
            pl.BlockSpec(memory_space=pltpu.SMEM),
            pl.BlockSpec(memory_space=pltpu.SMEM),
            pl.BlockSpec(memory_space=pltpu.SMEM),
        ],
        out_specs=[
            pl.BlockSpec((BR, 1024), lambda i: (i, 0)),
            pl.BlockSpec((BR, 1024), lambda i: (i, 0)),
        ],
        out_shape=[
            jax.ShapeDtypeStruct((1024, 1024), jnp.float32),
            jax.ShapeDtypeStruct((1024, 1024), jnp.float32),
        ],
    )(u2d, u2d, u2d, u2d, u2d, psum, wmat, profile, rp, sp)
    return out1, out2


def kernel(positions, motifs_prob, profile, replication_prob, short_patch_ber_prob):
    probs_b = jnp.broadcast_to(motifs_prob[:, None], (NM, 16))

    u, psum = _sc_scatter(positions, probs_b)
    out1, out2 = _tc_dense(u, psum, profile,
                           replication_prob, short_patch_ber_prob)
    return (out1.reshape(-1)[:SEQ], out2.reshape(-1)[:SEQ])
